# baseline (speedup 1.0000x reference)
import functools

import jax
import jax.numpy as jnp
from jax import lax
from jax.experimental import pallas as pl
from jax.experimental.pallas import tpu as pltpu
from jax.experimental.pallas import tpu_sc as plsc

_F32 = jnp.float32


def _tc_body(ids_ref, embt_ref, pw_ref, b_ref, log_ref):
    lblk = ids_ref.shape[0]
    nb = ids_ref.shape[1]

    embt = embt_ref[...]
    ltabt = jnp.dot(pw_ref[...], embt, preferred_element_type=_F32) + b_ref[...]

    iota_v = jax.lax.broadcasted_iota(jnp.int32, (16, nb), 0)
    for l in range(lblk):
        idrow = ids_ref[pl.ds(l, 1), :]
        oh = (jnp.broadcast_to(idrow, (16, nb)) == iota_v).astype(_F32)
        log_ref[l, :, :] = jnp.dot(ltabt, oh, preferred_element_type=_F32)


def _sc_body(b_per_w, seq, ids_hbm, lut_hbm, out_hbm, ids_v0, ids_v1, lut_v,
             out_v0, out_v1, isem0, isem1, sem0, sem1):
    nc = 2
    wid = lax.axis_index("s") * nc + lax.axis_index("c")
    b0 = wid * b_per_w
    ngrp = b_per_w // 16
    nsup = seq // 8

    pltpu.sync_copy(lut_hbm, lut_v)
    lrows = [lut_v[pl.ds(16 * d, 16)] for d in range(16)]
    obufs = (out_v0, out_v1)
    osems = (sem0, sem1)
    ibufs = (ids_v0, ids_v1)
    isems = (isem0, isem1)

    def ids_copy(blk4, j):
        return pltpu.make_async_copy(
            ids_hbm.at[pl.ds(blk4 * 4, 4), pl.ds(b0, b_per_w)],
            ibufs[j], isems[j])

    def out_copy(blk4, j):
        return pltpu.make_async_copy(
            obufs[j],
            out_hbm.at[pl.ds(blk4 * 4, 4), :, pl.ds(b0, b_per_w)],
            osems[j])

    def compute(ids_v, b4, out_v):
        def g_body(g, c):
            idvec = ids_v[b4, pl.ds(g * 16, 16)]
            for d in range(16):
                out_v[b4, d, pl.ds(g * 16, 16)] = lrows[d].at[idvec].get(
                    mode="promise_in_bounds")
            return c

        lax.fori_loop(0, ngrp, g_body, 0, unroll=4)

    ids_copy(0, 0).start()

    def sup_body(i, carry):
        for j in range(2):
            blk4 = 2 * i + j
            ids_copy(blk4, j).wait()

            @pl.when(blk4 + 1 < 2 * nsup)
            def _prefetch():
                ids_copy(blk4 + 1, 1 - j).start()

            @pl.when(i > 0)
            def _wait_prev():
                out_copy(blk4, j).wait()

            for b4 in range(4):
                compute(ibufs[j], b4, obufs[j])
            out_copy(blk4, j).start()
        return carry

    lax.fori_loop(0, nsup, sup_body, 0, unroll=False)
    for j in range(2):
        out_copy(2 * nsup - 2 + j, j).wait()


def kernel(input_ids, embed_table, proj_w, proj_b):
    bsz, seq = input_ids.shape
    d = embed_table.shape[1]
    ids_t = input_ids.T.astype(jnp.int32)
    embt = embed_table.T
    b_col = proj_b.reshape(d, 1)

    lblk = 8
    grid = seq // lblk

    log_t, = pl.pallas_call(
        _tc_body,
        grid=(grid,),
        in_specs=[
            pl.BlockSpec((lblk, bsz), lambda i: (i, 0)),
            pl.BlockSpec((d, d), lambda i: (0, 0)),
            pl.BlockSpec((d, d), lambda i: (0, 0)),
            pl.BlockSpec((d, 1), lambda i: (0, 0)),
        ],
        out_specs=[
            pl.BlockSpec((lblk, d, bsz), lambda i: (i, 0, 0)),
        ],
        out_shape=[
            jax.ShapeDtypeStruct((seq, d, bsz), _F32),
        ],
    )(ids_t, embt, proj_w, b_col)

    info = plsc.get_sparse_core_info()
    nw = info.num_cores * info.num_subcores
    b_per_w = bsz // nw
    mesh = plsc.VectorSubcoreMesh(core_axis_name="c", subcore_axis_name="s")
    lut = embt.reshape(d * d)

    sc_fn = pl.kernel(
        functools.partial(_sc_body, b_per_w, seq),
        mesh=mesh,
        out_type=jax.ShapeDtypeStruct((seq, d, bsz), _F32),
        scratch_types=[
            pltpu.VMEM((4, b_per_w), jnp.int32),
            pltpu.VMEM((4, b_per_w), jnp.int32),
            pltpu.VMEM((d * d,), _F32),
            pltpu.VMEM((4, d, b_per_w), _F32),
            pltpu.VMEM((4, d, b_per_w), _F32),
            pltpu.SemaphoreType.DMA,
            pltpu.SemaphoreType.DMA,
            pltpu.SemaphoreType.DMA,
            pltpu.SemaphoreType.DMA,
        ],
    )
    hid_t = sc_fn(ids_t, lut)

    hidden = jnp.transpose(hid_t, (2, 0, 1))
    logits = jnp.transpose(log_t, (2, 0, 1))
    return (logits, hidden)


# speedup vs baseline: 1.0195x; 1.0195x over previous
import functools

import jax
import jax.numpy as jnp
from jax import lax
from jax.experimental import pallas as pl
from jax.experimental.pallas import tpu as pltpu
from jax.experimental.pallas import tpu_sc as plsc

_F32 = jnp.float32


def _tc_body(ids_ref, embt_ref, pw_ref, b_ref, log_ref):
    lblk = ids_ref.shape[0]
    nb = ids_ref.shape[1]

    embt = embt_ref[...]
    ltabt = jnp.dot(pw_ref[...], embt, preferred_element_type=_F32) + b_ref[...]

    iota_v = jax.lax.broadcasted_iota(jnp.int32, (16, nb), 0)
    for l in range(lblk):
        idrow = ids_ref[pl.ds(l, 1), :]
        oh = (jnp.broadcast_to(idrow, (16, nb)) == iota_v).astype(_F32)
        log_ref[l, :, :] = jnp.dot(ltabt, oh, preferred_element_type=_F32)


def _sc_body(b_per_w, seq, ids_hbm, lut_hbm, out_hbm, ids_v0, ids_v1, lut_v,
             out_v0, out_v1, isem0, isem1, sem0, sem1):
    nc = 2
    wid = lax.axis_index("s") * nc + lax.axis_index("c")
    b0 = wid * b_per_w
    ngrp = b_per_w // 16
    nsup = seq // 8

    pltpu.sync_copy(lut_hbm, lut_v)
    lrows = [lut_v[pl.ds(16 * d, 16)] for d in range(16)]
    obufs = (out_v0, out_v1)
    osems = (sem0, sem1)
    ibufs = (ids_v0, ids_v1)
    isems = (isem0, isem1)

    def ids_copy(blk4, j):
        return pltpu.make_async_copy(
            ids_hbm.at[pl.ds(blk4 * 4, 4), pl.ds(b0, b_per_w)],
            ibufs[j], isems[j])

    def out_copy(blk4, j):
        return pltpu.make_async_copy(
            obufs[j],
            out_hbm.at[pl.ds(blk4 * 4, 4), :, pl.ds(b0, b_per_w)],
            osems[j])

    def compute(ids_v, b4, out_v):
        def g_body(g, c):
            idvec = ids_v[b4, pl.ds(g * 16, 16)]
            for d in range(16):
                out_v[b4, d, pl.ds(g * 16, 16)] = lrows[d].at[idvec].get(
                    mode="promise_in_bounds")
            return c

        lax.fori_loop(0, ngrp, g_body, 0, unroll=8)

    ids_copy(0, 0).start()

    def sup_body(i, carry):
        for j in range(2):
            blk4 = 2 * i + j
            ids_copy(blk4, j).wait()

            @pl.when(blk4 + 1 < 2 * nsup)
            def _prefetch():
                ids_copy(blk4 + 1, 1 - j).start()

            @pl.when(i > 0)
            def _wait_prev():
                out_copy(blk4, j).wait()

            for b4 in range(4):
                compute(ibufs[j], b4, obufs[j])
            out_copy(blk4, j).start()
        return carry

    lax.fori_loop(0, nsup, sup_body, 0, unroll=False)
    for j in range(2):
        out_copy(2 * nsup - 2 + j, j).wait()


def kernel(input_ids, embed_table, proj_w, proj_b):
    bsz, seq = input_ids.shape
    d = embed_table.shape[1]
    ids_t = input_ids.T.astype(jnp.int32)
    embt = embed_table.T
    b_col = proj_b.reshape(d, 1)

    lblk = 8
    grid = seq // lblk

    log_t, = pl.pallas_call(
        _tc_body,
        grid=(grid,),
        in_specs=[
            pl.BlockSpec((lblk, bsz), lambda i: (i, 0)),
            pl.BlockSpec((d, d), lambda i: (0, 0)),
            pl.BlockSpec((d, d), lambda i: (0, 0)),
            pl.BlockSpec((d, 1), lambda i: (0, 0)),
        ],
        out_specs=[
            pl.BlockSpec((lblk, d, bsz), lambda i: (i, 0, 0)),
        ],
        out_shape=[
            jax.ShapeDtypeStruct((seq, d, bsz), _F32),
        ],
    )(ids_t, embt, proj_w, b_col)

    info = plsc.get_sparse_core_info()
    nw = info.num_cores * info.num_subcores
    b_per_w = bsz // nw
    mesh = plsc.VectorSubcoreMesh(core_axis_name="c", subcore_axis_name="s")
    lut = embt.reshape(d * d)

    sc_fn = pl.kernel(
        functools.partial(_sc_body, b_per_w, seq),
        mesh=mesh,
        out_type=jax.ShapeDtypeStruct((seq, d, bsz), _F32),
        scratch_types=[
            pltpu.VMEM((4, b_per_w), jnp.int32),
            pltpu.VMEM((4, b_per_w), jnp.int32),
            pltpu.VMEM((d * d,), _F32),
            pltpu.VMEM((4, d, b_per_w), _F32),
            pltpu.VMEM((4, d, b_per_w), _F32),
            pltpu.SemaphoreType.DMA,
            pltpu.SemaphoreType.DMA,
            pltpu.SemaphoreType.DMA,
            pltpu.SemaphoreType.DMA,
        ],
    )
    hid_t = sc_fn(ids_t, lut)

    hidden = jnp.transpose(hid_t, (2, 0, 1))
    logits = jnp.transpose(log_t, (2, 0, 1))
    return (logits, hidden)
